# Initial kernel scaffold; baseline (speedup 1.0000x reference)
import functools

import jax
import jax.numpy as jnp
from jax import lax
from jax.experimental import pallas as pl
from jax.experimental.pallas import tpu as pltpu
from jax.experimental.pallas import tpu_sc as plsc

_CHUNK = 128
_CPS = 8
_SUPER = _CHUNK * _CPS
_NBUF = 2


@functools.lru_cache(maxsize=None)
def _make_kernel(total, emb):
    mesh = plsc.VectorSubcoreMesh(core_axis_name="c", subcore_axis_name="s")
    nc = mesh.num_cores
    nw = nc * mesh.num_subcores
    per_w = total // nw
    n_super = per_w // _SUPER
    assert per_w % _SUPER == 0 and n_super % _NBUF == 0

    @functools.partial(
        pl.kernel,
        out_type=jax.ShapeDtypeStruct((total, emb), jnp.float32),
        mesh=mesh,
        scratch_types=[
            pltpu.VMEM((_NBUF, _CPS, _CHUNK), jnp.int32),
            pltpu.VMEM((_NBUF, _SUPER, emb), jnp.float32),
            pltpu.SemaphoreType.DMA,
            pltpu.SemaphoreType.DMA,
        ],
    )
    def k(idx_hbm, table_hbm, out_hbm, idx_v, rows_v, gsem, osem):
        wid = lax.axis_index("s") * nc + lax.axis_index("c")
        base_chunk = wid * (per_w // _CHUNK)
        base_row = wid * per_w

        @pl.loop(0, n_super, step=_NBUF)
        def body(s0):
            for b in range(_NBUF):
                s = s0 + b

                @pl.when(s >= _NBUF)
                def _():
                    pltpu.make_async_copy(
                        out_hbm.at[pl.ds(base_row, _SUPER)],
                        rows_v.at[b],
                        osem,
                    ).wait()

                pltpu.sync_copy(
                    idx_hbm.at[pl.ds(base_chunk + s * _CPS, _CPS)],
                    idx_v.at[b],
                )
                descs = [
                    pltpu.async_copy(
                        table_hbm.at[idx_v.at[b, j]],
                        rows_v.at[b, pl.ds(j * _CHUNK, _CHUNK)],
                        gsem,
                    )
                    for j in range(_CPS)
                ]
                for d in descs:
                    d.wait()
                pltpu.async_copy(
                    rows_v.at[b],
                    out_hbm.at[pl.ds(base_row + s * _SUPER, _SUPER)],
                    osem,
                )

        for b in range(_NBUF):
            pltpu.make_async_copy(
                out_hbm.at[pl.ds(base_row, _SUPER)],
                rows_v.at[b],
                osem,
            ).wait()

    return k


def kernel(indices, table):
    bsz, hist = indices.shape
    _, emb = table.shape
    total = bsz * hist
    idx2d = indices.reshape(total // _CHUNK, _CHUNK).astype(jnp.int32)
    out = _make_kernel(total, emb)(idx2d, table)
    return out.reshape(bsz, hist, emb)


# speedup vs baseline: 4.4404x; 4.4404x over previous
import functools

import jax
import jax.numpy as jnp
from jax import lax
from jax.experimental import pallas as pl
from jax.experimental.pallas import tpu as pltpu
from jax.experimental.pallas import tpu_sc as plsc

_CHUNK = 128
_CPS = 8
_SUPER = _CHUNK * _CPS
_NBUF = 2


@functools.lru_cache(maxsize=None)
def _make_kernel(total, emb):
    mesh = plsc.VectorSubcoreMesh(core_axis_name="c", subcore_axis_name="s")
    nc = mesh.num_cores
    nw = nc * mesh.num_subcores
    per_w = total // nw
    n_super = per_w // _SUPER
    assert per_w % _SUPER == 0 and n_super % _NBUF == 0

    @functools.partial(
        pl.kernel,
        out_type=jax.ShapeDtypeStruct((total, emb), jnp.float32),
        mesh=mesh,
        compiler_params=pltpu.CompilerParams(use_tc_tiling_on_sc=False),
        scratch_types=[
            pltpu.VMEM((_NBUF, _CPS, _CHUNK), jnp.int32),
            pltpu.VMEM((_NBUF, _SUPER, emb), jnp.float32),
            pltpu.SemaphoreType.DMA,
            pltpu.SemaphoreType.DMA,
        ],
    )
    def k(idx_hbm, table_hbm, out_hbm, idx_v, rows_v, gsem, osem):
        wid = lax.axis_index("s") * nc + lax.axis_index("c")
        base_chunk = wid * (per_w // _CHUNK)
        base_row = wid * per_w

        @pl.loop(0, n_super, step=_NBUF)
        def body(s0):
            for b in range(_NBUF):
                s = s0 + b

                @pl.when(s >= _NBUF)
                def _():
                    pltpu.make_async_copy(
                        out_hbm.at[pl.ds(base_row, _SUPER)],
                        rows_v.at[b],
                        osem,
                    ).wait()

                pltpu.sync_copy(
                    idx_hbm.at[pl.ds(base_chunk + s * _CPS, _CPS)],
                    idx_v.at[b],
                )
                descs = [
                    pltpu.async_copy(
                        table_hbm.at[idx_v.at[b, j]],
                        rows_v.at[b, pl.ds(j * _CHUNK, _CHUNK)],
                        gsem,
                    )
                    for j in range(_CPS)
                ]
                for d in descs:
                    d.wait()
                pltpu.async_copy(
                    rows_v.at[b],
                    out_hbm.at[pl.ds(base_row + s * _SUPER, _SUPER)],
                    osem,
                )

        for b in range(_NBUF):
            pltpu.make_async_copy(
                out_hbm.at[pl.ds(base_row, _SUPER)],
                rows_v.at[b],
                osem,
            ).wait()

    return k


def kernel(indices, table):
    bsz, hist = indices.shape
    _, emb = table.shape
    total = bsz * hist
    idx2d = indices.reshape(total // _CHUNK, _CHUNK).astype(jnp.int32)
    out = _make_kernel(total, emb)(idx2d, table)
    return out.reshape(bsz, hist, emb)


# speedup vs baseline: 6.8486x; 1.5423x over previous
import functools

import jax
import jax.numpy as jnp
from jax import lax
from jax.experimental import pallas as pl
from jax.experimental.pallas import tpu as pltpu
from jax.experimental.pallas import tpu_sc as plsc

_CHUNK = 128
_CPS = 2
_SUPER = _CHUNK * _CPS
_NBUF = 2
_IBLK = 16
_GRP = 4
_PAD = 128


@functools.lru_cache(maxsize=None)
def _make_format_kernel(ncols, emb):
    mesh = plsc.VectorSubcoreMesh(core_axis_name="c", subcore_axis_name="s")
    nc = mesh.num_cores
    nw = nc * mesh.num_subcores
    n_full = ncols // _PAD
    n_blk = n_full + 1
    n_grp = n_full // _GRP
    assert n_grp * _GRP == n_full
    per_w = n_grp // nw
    n_extra = n_grp - per_w * nw

    @functools.partial(
        pl.kernel,
        out_type=jax.ShapeDtypeStruct((n_blk * _PAD * _PAD,), jnp.float32),
        mesh=mesh,
        compiler_params=pltpu.CompilerParams(
            use_tc_tiling_on_sc=True, needs_layout_passes=False),
        scratch_types=[
            pltpu.VMEM((_NBUF, _GRP, 64, _PAD), jnp.float32),
            pltpu.VMEM((_NBUF * _PAD * _PAD,), jnp.float32),
            pltpu.SemaphoreType.DMA,
            pltpu.SemaphoreType.DMA,
        ],
    )
    def k(tab_hbm, tail_hbm, out_hbm, src_v, trans_v, rsem, wsem):
        wid = lax.axis_index("s") * nc + lax.axis_index("c")
        cnt = jnp.where(wid < n_extra, per_w + 1, per_w)
        start = wid * per_w + jnp.minimum(wid, n_extra)
        lane = lax.iota(jnp.int32, 16)
        rowmul = [lane * 128 + 2048 * kk for kk in range(8)]

        @pl.loop(0, per_w + 1, step=_NBUF)
        def body(s0):
            for b in range(_NBUF):
                s = s0 + b

                @pl.when(s < cnt)
                def _():
                    gidx = start + s

                    rdescs = [
                        pltpu.async_copy(
                            tab_hbm.at[:, pl.ds((gidx * _GRP + j) * _PAD,
                                                _PAD)],
                            src_v.at[b, j, pl.ds(0, emb)],
                            rsem,
                        )
                        for j in range(_GRP)
                    ]
                    for d in rdescs:
                        d.wait()

                    for j in range(_GRP):
                        tb = j % _NBUF

                        if j >= _NBUF:
                            pltpu.make_async_copy(
                                out_hbm.at[pl.ds(0, _PAD * _PAD)],
                                trans_v.at[pl.ds(tb * _PAD * _PAD, _PAD * _PAD)],
                                wsem,
                            ).wait()
                        else:
                            @pl.when(s >= 1)
                            def _():
                                pltpu.make_async_copy(
                                    out_hbm.at[pl.ds(0, _PAD * _PAD)],
                                    trans_v.at[pl.ds(tb * _PAD * _PAD, _PAD * _PAD)],
                                    wsem,
                                ).wait()

                        @plsc.parallel_loop(0, 64, step=2, unroll=2)
                        def tr(c0):
                            for dc in range(2):
                                c = c0 + dc
                                cvec = jnp.full((16,), 0, jnp.int32) + c
                                for kk in range(8):
                                    v = src_v[b, j, c, pl.ds(16 * kk, 16)]
                                    plsc.store_scatter(
                                        trans_v,
                                        [rowmul[kk] + cvec
                                         + tb * _PAD * _PAD], v)

                        pltpu.async_copy(
                            trans_v.at[pl.ds(tb * _PAD * _PAD,
                                             _PAD * _PAD)],
                            out_hbm.at[pl.ds(
                                (gidx * _GRP + j) * _PAD * _PAD,
                                _PAD * _PAD)],
                            wsem,
                        )

        for b in range(_NBUF):
            pltpu.make_async_copy(
                out_hbm.at[pl.ds(0, _PAD * _PAD)],
                trans_v.at[pl.ds(b * _PAD * _PAD, _PAD * _PAD)],
                wsem,
            ).wait()

        @pl.when(wid == nw - 1)
        def _():
            pltpu.sync_copy(tail_hbm,
                            trans_v.at[pl.ds(0, _PAD * _PAD)])
            pltpu.sync_copy(
                trans_v.at[pl.ds(0, _PAD * _PAD)],
                out_hbm.at[pl.ds(n_full * _PAD * _PAD, _PAD * _PAD)])

    return k


@functools.lru_cache(maxsize=None)
def _make_gather_kernel(total, nrows_pad):
    mesh = plsc.VectorSubcoreMesh(core_axis_name="c", subcore_axis_name="s")
    nc = mesh.num_cores
    nw = nc * mesh.num_subcores
    per_w = total // nw
    n_super = per_w // _SUPER
    assert per_w % _SUPER == 0 and n_super % _NBUF == 0

    assert n_super % _IBLK == 0

    @functools.partial(
        pl.kernel,
        out_type=jax.ShapeDtypeStruct((total, _PAD), jnp.float32),
        mesh=mesh,
        compiler_params=pltpu.CompilerParams(use_tc_tiling_on_sc=True),
        scratch_types=[
            pltpu.VMEM((_IBLK * _CPS, _CHUNK), jnp.int32),
            pltpu.VMEM((_NBUF, _SUPER, _PAD), jnp.float32),
            pltpu.SemaphoreType.DMA,
            pltpu.SemaphoreType.DMA,
        ],
    )
    def k(idx_hbm, table_hbm, out_hbm, idx_v, rows_v, gsem, osem):
        wid = lax.axis_index("s") * nc + lax.axis_index("c")
        base_chunk = wid * (per_w // _CHUNK)
        base_row = wid * per_w

        @pl.loop(0, n_super // _IBLK)
        def iblk(t):
            s_base = t * _IBLK
            pltpu.sync_copy(
                idx_hbm.at[pl.ds(base_chunk + s_base * _CPS, _IBLK * _CPS)],
                idx_v,
            )

            @pl.loop(0, _IBLK, step=_NBUF)
            def body(u0):
                for b in range(_NBUF):
                    u = u0 + b
                    s = s_base + u

                    @pl.when(s >= _NBUF)
                    def _():
                        pltpu.make_async_copy(
                            out_hbm.at[pl.ds(base_row, _SUPER)],
                            rows_v.at[b],
                            osem,
                        ).wait()

                    descs = [
                        pltpu.async_copy(
                            table_hbm.at[idx_v.at[u * _CPS + j]],
                            rows_v.at[b, pl.ds(j * _CHUNK, _CHUNK)],
                            gsem,
                        )
                        for j in range(_CPS)
                    ]
                    for d in descs:
                        d.wait()
                    pltpu.async_copy(
                        rows_v.at[b],
                        out_hbm.at[pl.ds(base_row + s * _SUPER, _SUPER)],
                        osem,
                    )

        for b in range(_NBUF):
            pltpu.make_async_copy(
                out_hbm.at[pl.ds(base_row, _SUPER)],
                rows_v.at[b],
                osem,
            ).wait()

    return k


def kernel(indices, table):
    bsz, hist = indices.shape
    nrows, emb = table.shape
    total = bsz * hist
    idx2d = indices.reshape(total // _CHUNK, _CHUNK).astype(jnp.int32)
    n_tail = nrows % _PAD
    tail_src = jnp.pad(table[nrows - n_tail :],
                       ((0, _PAD - n_tail), (0, _PAD - emb)))
    tab_flat = _make_format_kernel(nrows, emb)(
        table.T, tail_src.reshape(_PAD * _PAD))
    table_pad = tab_flat.reshape(-1, _PAD)
    out = _make_gather_kernel(total, table_pad.shape[0])(idx2d, table_pad)
    return out[:, :emb].reshape(bsz, hist, emb)
